# baseline (speedup 1.0000x reference)
import functools

import jax
import jax.numpy as jnp
from jax import lax
from jax.experimental import pallas as pl
from jax.experimental.pallas import tpu as pltpu
from jax.experimental.pallas import tpu_sc as plsc

NUM_FIELDS = 26
VOCAB = 100000
EMB_DIM = 16
BATCH = 4096

_NC = 2
_NS = 16
_NW = _NC * _NS
_CHUNKS = BATCH // 128
_LAG = 48
_SPLITS = (8, 6, 6, 6)


def _make_body(n_fields):
    rows = n_fields * EMB_DIM
    r_per_w = rows // _NW
    n_dma = r_per_w * _CHUNKS
    n_stage = 2

    def body(tab_hbm, idx_hbm, out_hbm, idx_v, rows_v, sem):
        wid = lax.axis_index("s") * _NC + lax.axis_index("c")
        r0 = wid * r_per_w
        f0 = r0 // EMB_DIM
        f1 = (r0 + r_per_w - 1) // EMB_DIM
        pltpu.sync_copy(idx_hbm.at[f0], idx_v.at[0])
        pltpu.sync_copy(idx_hbm.at[f1], idx_v.at[1])

        def slices(i):
            row = i // _CHUNKS
            chunk = i - row * _CHUNKS
            r = r0 + row
            f = r // EMB_DIM
            d = r - f * EMB_DIM
            src = tab_hbm.at[f].at[d].at[idx_v.at[f - f0].at[chunk]]
            dst = rows_v.at[pl.ds(i * 128, 128)]
            return src, dst

        def fire_body(i, carry):
            src, dst = slices(i)
            pltpu.async_copy(src, dst, sem)

            @pl.when(i >= _LAG)
            def _():
                src2, dst2 = slices(i - _LAG)
                pltpu.make_async_copy(src2, dst2, sem).wait()

            return carry

        lax.fori_loop(0, n_dma, fire_body, 0)

        def drain_body(i, carry):
            src2, dst2 = slices(i)
            pltpu.make_async_copy(src2, dst2, sem).wait()
            return carry

        lax.fori_loop(n_dma - _LAG, n_dma, drain_body, 0)

        pltpu.sync_copy(rows_v, out_hbm.at[pl.ds(wid * n_dma * 128, n_dma * 128)])

    return body, rows, n_dma


@jax.jit
def _impl(indices, tables):
    tabT = jnp.transpose(tables, (0, 2, 1))
    idxT = indices.T.reshape(NUM_FIELDS, _CHUNKS, 128)
    mesh = plsc.VectorSubcoreMesh(core_axis_name="c", subcore_axis_name="s")
    outs = []
    f_base = 0
    for n_fields in _SPLITS:
        body, rows, n_dma = _make_body(n_fields)
        run = pl.kernel(
            body,
            out_type=jax.ShapeDtypeStruct((rows * BATCH,), jnp.float32),
            mesh=mesh,
            compiler_params=pltpu.CompilerParams(use_tc_tiling_on_sc=False),
            scratch_types=[
                pltpu.VMEM((2, _CHUNKS, 128), jnp.int32),
                pltpu.VMEM((n_dma * 128,), jnp.float32),
                pltpu.SemaphoreType.DMA,
            ],
        )
        outs.append(
            run(tabT[f_base:f_base + n_fields], idxT[f_base:f_base + n_fields])
        )
        f_base += n_fields
    out = jnp.concatenate(outs)
    return out.reshape(NUM_FIELDS * EMB_DIM, BATCH).T.reshape(
        BATCH, NUM_FIELDS * EMB_DIM)


def kernel(indices, tables):
    return _impl(indices, tables)


# speedup vs baseline: 1.0024x; 1.0024x over previous
import functools

import jax
import jax.numpy as jnp
from jax import lax
from jax.experimental import pallas as pl
from jax.experimental.pallas import tpu as pltpu
from jax.experimental.pallas import tpu_sc as plsc

NUM_FIELDS = 26
VOCAB = 100000
EMB_DIM = 16
BATCH = 4096

_NC = 2
_NS = 16
_NW = _NC * _NS
_CHUNKS = BATCH // 128
_LAG = 48
_SPLITS = (16, 10)


def _make_body(n_fields):
    rows = n_fields * EMB_DIM
    r_per_w = rows // _NW
    n_dma = r_per_w * _CHUNKS
    n_stage = 2

    def body(tab_hbm, idx_hbm, out_hbm, idx_v, rows_v, sem):
        wid = lax.axis_index("s") * _NC + lax.axis_index("c")
        r0 = wid * r_per_w
        f0 = r0 // EMB_DIM
        f1 = (r0 + r_per_w - 1) // EMB_DIM
        pltpu.sync_copy(idx_hbm.at[f0], idx_v.at[0])
        pltpu.sync_copy(idx_hbm.at[f1], idx_v.at[1])

        def slices(i):
            row = i // _CHUNKS
            chunk = i - row * _CHUNKS
            r = r0 + row
            f = r // EMB_DIM
            d = r - f * EMB_DIM
            src = tab_hbm.at[f].at[d].at[idx_v.at[f - f0].at[chunk]]
            dst = rows_v.at[pl.ds(i * 128, 128)]
            return src, dst

        def fire_body(i, carry):
            src, dst = slices(i)
            pltpu.async_copy(src, dst, sem)

            @pl.when(i >= _LAG)
            def _():
                src2, dst2 = slices(i - _LAG)
                pltpu.make_async_copy(src2, dst2, sem).wait()

            return carry

        lax.fori_loop(0, n_dma, fire_body, 0)

        def drain_body(i, carry):
            src2, dst2 = slices(i)
            pltpu.make_async_copy(src2, dst2, sem).wait()
            return carry

        lax.fori_loop(n_dma - _LAG, n_dma, drain_body, 0)

        pltpu.sync_copy(rows_v, out_hbm.at[pl.ds(wid * n_dma * 128, n_dma * 128)])

    return body, rows, n_dma


@jax.jit
def _impl(indices, tables):
    tabT = jnp.transpose(tables, (0, 2, 1))
    idxT = indices.T.reshape(NUM_FIELDS, _CHUNKS, 128)
    mesh = plsc.VectorSubcoreMesh(core_axis_name="c", subcore_axis_name="s")
    outs = []
    f_base = 0
    for n_fields in _SPLITS:
        body, rows, n_dma = _make_body(n_fields)
        run = pl.kernel(
            body,
            out_type=jax.ShapeDtypeStruct((rows * BATCH,), jnp.float32),
            mesh=mesh,
            compiler_params=pltpu.CompilerParams(use_tc_tiling_on_sc=False),
            scratch_types=[
                pltpu.VMEM((2, _CHUNKS, 128), jnp.int32),
                pltpu.VMEM((n_dma * 128,), jnp.float32),
                pltpu.SemaphoreType.DMA,
            ],
        )
        outs.append(
            run(tabT[f_base:f_base + n_fields], idxT[f_base:f_base + n_fields])
        )
        f_base += n_fields
    out = jnp.concatenate(outs)
    return out.reshape(NUM_FIELDS * EMB_DIM, BATCH).T.reshape(
        BATCH, NUM_FIELDS * EMB_DIM)


def kernel(indices, tables):
    return _impl(indices, tables)
